# baseline (speedup 1.0000x reference)
import functools

import jax
import jax.numpy as jnp
from jax import lax
from jax.experimental import pallas as pl
from jax.experimental.pallas import tpu as pltpu
from jax.experimental.pallas import tpu_sc as plsc

NUM_SPECIAL = 1000
PAD_IDX = NUM_SPECIAL
VOCAB = NUM_SPECIAL + 1
DIM = 64
BATCH, SEQ = 4096, 200

B = BATCH * SEQ
CHUNK = 128
N_CHUNKS = B // CHUNK
NC, NS = 2, 16
NW = NC * NS
CHUNKS_PER_W = N_CHUNKS // NW
L = 16
COLS = DIM // L
BLK_S = 3
BLK_T = 2
BLOCK = BLK_S + BLK_T
NBLK = CHUNKS_PER_W // BLOCK
IDXH = CHUNKS_PER_W // 2


def _emb_body(idx_hbm, tbl_hbm, out_hbm,
              tbl_s, tbl_l, idx_v, sbuf, tbuf, gsem, ssem, tsem):
    sid = lax.axis_index("s")
    wid = sid * NC + lax.axis_index("c")
    c0 = wid * CHUNKS_PER_W

    @pl.when(sid == 0)
    def _():
        pltpu.sync_copy(tbl_hbm, tbl_s)

    pltpu.sync_copy(tbl_hbm, tbl_l)
    pltpu.sync_copy(idx_hbm.at[pl.ds(c0, IDXH)], idx_v)
    plsc.subcore_barrier()

    def out_at(a):
        return out_hbm.at[pl.ds((c0 + a) * CHUNK, CHUNK)]

    def tec_gather(t, b):
        def group(q, _):
            ivec = idx_v[lax.rem(t, IDXH), pl.ds(q * L, L)]
            for j in range(L):
                row = ivec[j]
                for c in range(COLS):
                    tbuf[b, q * L + j, pl.ds(c * L, L)] = (
                        tbl_l[row, pl.ds(c * L, L)])
            return 0

        lax.fori_loop(0, CHUNK // L, group, 0)

    def block(k, _):
        base = k * BLOCK

        @pl.when(k == NBLK // 2)
        def _():
            pltpu.sync_copy(idx_hbm.at[pl.ds(c0 + IDXH, IDXH)], idx_v)

        for j in range(BLK_S):
            a = base + j

            @pl.when(k > 0)
            def _():
                pltpu.make_async_copy(sbuf.at[j], out_at(a - BLOCK),
                                      ssem.at[j]).wait()

            pltpu.async_copy(tbl_s.at[idx_v.at[lax.rem(a, IDXH)]],
                             sbuf.at[j], gsem.at[j])

        for b in range(BLK_T):
            t = base + BLK_S + b

            @pl.when(k > 0)
            def _():
                pltpu.make_async_copy(tbuf.at[b], out_at(t - BLOCK),
                                      tsem.at[b]).wait()

            tec_gather(t, b)
            pltpu.async_copy(tbuf.at[b], out_at(t), tsem.at[b])

        for j in range(BLK_S):
            a = base + j
            pltpu.make_async_copy(tbl_s.at[idx_v.at[lax.rem(a, IDXH)]],
                                  sbuf.at[j], gsem.at[j]).wait()
            pltpu.async_copy(sbuf.at[j], out_at(a), ssem.at[j])
        return 0

    lax.fori_loop(0, NBLK, block, 0)

    last = (NBLK - 1) * BLOCK
    for j in range(BLK_S):
        pltpu.make_async_copy(sbuf.at[j], out_at(last + j), ssem.at[j]).wait()
    for b in range(BLK_T):
        pltpu.make_async_copy(tbuf.at[b], out_at(last + BLK_S + b),
                              tsem.at[b]).wait()


@jax.jit
def _emb_lookup(idx2d, embs):
    mesh = plsc.VectorSubcoreMesh(core_axis_name="c", subcore_axis_name="s")
    f = pl.kernel(
        _emb_body,
        out_type=jax.ShapeDtypeStruct((B, DIM), jnp.float32),
        mesh=mesh,
        scratch_types=[
            pltpu.VMEM_SHARED((VOCAB, DIM), jnp.float32),
            pltpu.VMEM((VOCAB, DIM), jnp.float32),
            pltpu.VMEM((IDXH, CHUNK), jnp.int32),
            pltpu.VMEM((BLK_S, CHUNK, DIM), jnp.float32),
            pltpu.VMEM((BLK_T, CHUNK, DIM), jnp.float32),
            pltpu.SemaphoreType.DMA((BLK_S,)),
            pltpu.SemaphoreType.DMA((BLK_S,)),
            pltpu.SemaphoreType.DMA((BLK_T,)),
        ],
        compiler_params=pltpu.CompilerParams(use_tc_tiling_on_sc=False),
    )
    return f(idx2d, embs)


def kernel(inputs, embs):
    idx2d = inputs.reshape(N_CHUNKS, CHUNK)
    out = _emb_lookup(idx2d, embs)
    return out.reshape(BATCH, SEQ, DIM)


# speedup vs baseline: 1.0379x; 1.0379x over previous
import functools

import jax
import jax.numpy as jnp
from jax import lax
from jax.experimental import pallas as pl
from jax.experimental.pallas import tpu as pltpu
from jax.experimental.pallas import tpu_sc as plsc

NUM_SPECIAL = 1000
PAD_IDX = NUM_SPECIAL
VOCAB = NUM_SPECIAL + 1
DIM = 64
BATCH, SEQ = 4096, 200

B = BATCH * SEQ
CHUNK = 128
N_CHUNKS = B // CHUNK
NC, NS = 2, 16
NW = NC * NS
CHUNKS_PER_W = N_CHUNKS // NW
L = 16
COLS = DIM // L
BLK_S = 3
BLK_T = 2
BLOCK = BLK_S + BLK_T
NBLK = CHUNKS_PER_W // BLOCK
IDXH = CHUNKS_PER_W // 2


def _emb_body(idx_hbm, tbl_hbm, out_hbm,
              tbl_s, idx_v, sbuf, tbuf, gsem, ssem, hsem, tsem):
    sid = lax.axis_index("s")
    wid = sid * NC + lax.axis_index("c")
    c0 = wid * CHUNKS_PER_W

    @pl.when(sid == 0)
    def _():
        pltpu.sync_copy(tbl_hbm, tbl_s)

    pltpu.sync_copy(idx_hbm.at[pl.ds(c0, IDXH)], idx_v)
    plsc.subcore_barrier()

    def out_at(a):
        return out_hbm.at[pl.ds((c0 + a) * CHUNK, CHUNK)]

    def block(k, _):
        base = k * BLOCK

        @pl.when(k == NBLK // 2)
        def _():
            pltpu.sync_copy(idx_hbm.at[pl.ds(c0 + IDXH, IDXH)], idx_v)

        for j in range(BLK_S):
            a = base + j

            @pl.when(k > 0)
            def _():
                pltpu.make_async_copy(sbuf.at[j], out_at(a - BLOCK),
                                      ssem.at[j]).wait()

            pltpu.async_copy(tbl_s.at[idx_v.at[lax.rem(a, IDXH)]],
                             sbuf.at[j], gsem.at[j])

        for b in range(BLK_T):
            t = base + BLK_S + b

            @pl.when(k > 0)
            def _():
                pltpu.make_async_copy(tbuf.at[b], out_at(t - BLOCK),
                                      tsem.at[b]).wait()

            pltpu.async_copy(tbl_hbm.at[idx_v.at[lax.rem(t, IDXH)]],
                             tbuf.at[b], hsem.at[b])

        for j in range(BLK_S):
            a = base + j
            pltpu.make_async_copy(tbl_s.at[idx_v.at[lax.rem(a, IDXH)]],
                                  sbuf.at[j], gsem.at[j]).wait()
            pltpu.async_copy(sbuf.at[j], out_at(a), ssem.at[j])
        for b in range(BLK_T):
            t = base + BLK_S + b
            pltpu.make_async_copy(tbl_hbm.at[idx_v.at[lax.rem(t, IDXH)]],
                                  tbuf.at[b], hsem.at[b]).wait()
            pltpu.async_copy(tbuf.at[b], out_at(t), tsem.at[b])
        return 0

    lax.fori_loop(0, NBLK, block, 0)

    last = (NBLK - 1) * BLOCK
    for j in range(BLK_S):
        pltpu.make_async_copy(sbuf.at[j], out_at(last + j), ssem.at[j]).wait()
    for b in range(BLK_T):
        pltpu.make_async_copy(tbuf.at[b], out_at(last + BLK_S + b),
                              tsem.at[b]).wait()


@jax.jit
def _emb_lookup(idx2d, embs):
    mesh = plsc.VectorSubcoreMesh(core_axis_name="c", subcore_axis_name="s")
    f = pl.kernel(
        _emb_body,
        out_type=jax.ShapeDtypeStruct((B, DIM), jnp.float32),
        mesh=mesh,
        scratch_types=[
            pltpu.VMEM_SHARED((VOCAB, DIM), jnp.float32),
            pltpu.VMEM((IDXH, CHUNK), jnp.int32),
            pltpu.VMEM((BLK_S, CHUNK, DIM), jnp.float32),
            pltpu.VMEM((BLK_T, CHUNK, DIM), jnp.float32),
            pltpu.SemaphoreType.DMA((BLK_S,)),
            pltpu.SemaphoreType.DMA((BLK_S,)),
            pltpu.SemaphoreType.DMA((BLK_T,)),
            pltpu.SemaphoreType.DMA((BLK_T,)),
        ],
        compiler_params=pltpu.CompilerParams(use_tc_tiling_on_sc=False),
    )
    return f(idx2d, embs)


def kernel(inputs, embs):
    idx2d = inputs.reshape(N_CHUNKS, CHUNK)
    out = _emb_lookup(idx2d, embs)
    return out.reshape(BATCH, SEQ, DIM)


# speedup vs baseline: 1.2171x; 1.1726x over previous
import functools

import jax
import jax.numpy as jnp
from jax import lax
from jax.experimental import pallas as pl
from jax.experimental.pallas import tpu as pltpu
from jax.experimental.pallas import tpu_sc as plsc

NUM_SPECIAL = 1000
PAD_IDX = NUM_SPECIAL
VOCAB = NUM_SPECIAL + 1
DIM = 64
BATCH, SEQ = 4096, 200

B = BATCH * SEQ
CHUNK = 128
N_CHUNKS = B // CHUNK
NC, NS = 2, 16
NW = NC * NS
CHUNKS_PER_W = N_CHUNKS // NW
NBUF = 4
PREF = 2


def _emb_body(idx_hbm, tbl_hbm, out_hbm, tbl_v, idx_v, rows_v, gsem, ssem):
    wid = lax.axis_index("s") * NC + lax.axis_index("c")
    c0 = wid * CHUNKS_PER_W

    @pl.when(lax.axis_index("s") == 0)
    def _():
        pltpu.sync_copy(tbl_hbm, tbl_v)

    pltpu.sync_copy(idx_hbm.at[pl.ds(c0, CHUNKS_PER_W)], idx_v)
    plsc.subcore_barrier()

    def gather(g):
        slot = lax.rem(g, NBUF)
        pltpu.async_copy(tbl_v.at[idx_v.at[g]], rows_v.at[slot],
                         gsem.at[slot])

    for b in range(PREF):
        gather(b)

    def step(g, _):
        slot = lax.rem(g, NBUF)
        chunk = c0 + g

        pg = g + PREF

        @pl.when(pg < CHUNKS_PER_W)
        def _():
            pslot = lax.rem(pg, NBUF)

            @pl.when(pg >= NBUF)
            def _():
                pltpu.make_async_copy(
                    rows_v.at[pslot],
                    out_hbm.at[pl.ds((c0 + pg - NBUF) * CHUNK, CHUNK)],
                    ssem.at[pslot]).wait()

            gather(pg)

        pltpu.make_async_copy(tbl_v.at[idx_v.at[g]], rows_v.at[slot],
                              gsem.at[slot]).wait()
        pltpu.async_copy(rows_v.at[slot], out_hbm.at[pl.ds(chunk * CHUNK, CHUNK)],
                         ssem.at[slot])
        return 0

    lax.fori_loop(0, CHUNKS_PER_W, step, 0)

    def drain(g, _):
        slot = lax.rem(g, NBUF)
        chunk = c0 + g
        pltpu.make_async_copy(
            rows_v.at[slot], out_hbm.at[pl.ds(chunk * CHUNK, CHUNK)],
            ssem.at[slot]).wait()
        return 0

    lax.fori_loop(CHUNKS_PER_W - NBUF, CHUNKS_PER_W, drain, 0)


@jax.jit
def _emb_lookup(idx2d, embs):
    mesh = plsc.VectorSubcoreMesh(core_axis_name="c", subcore_axis_name="s")
    f = pl.kernel(
        _emb_body,
        out_type=jax.ShapeDtypeStruct((B, DIM), jnp.float32),
        mesh=mesh,
        scratch_types=[
            pltpu.VMEM_SHARED((VOCAB, DIM), jnp.float32),
            pltpu.VMEM((CHUNKS_PER_W, CHUNK), jnp.int32),
            pltpu.VMEM((NBUF, CHUNK, DIM), jnp.float32),
            pltpu.SemaphoreType.DMA((NBUF,)),
            pltpu.SemaphoreType.DMA((NBUF,)),
        ],
        compiler_params=pltpu.CompilerParams(use_tc_tiling_on_sc=False),
    )
    return f(idx2d, embs)


def kernel(inputs, embs):
    idx2d = inputs.reshape(N_CHUNKS, CHUNK)
    out = _emb_lookup(idx2d, embs)
    return out.reshape(BATCH, SEQ, DIM)
